# Initial kernel scaffold; baseline (speedup 1.0000x reference)
import jax
import jax.numpy as jnp
from jax.experimental import pallas as pl


def kernel(inputs, tables, W1_0, b1_0, W2_0, b2_0, W1_1, b1_1, W2_1, b2_1, W1_2, b1_2, W2_2, b2_2, Wd, bd):
    raise NotImplementedError("write your pallas kernel here")


# speedup vs baseline: 10.4309x; 10.4309x over previous
import functools

import jax
import jax.numpy as jnp
from jax.experimental import pallas as pl
from jax.experimental.pallas import tpu as pltpu
from jax.experimental.pallas import tpu_sc as plsc

B = 4096
F = 26
V = 1000
D = 128
L = F * D

_GATHER_WINDOW = 128


def _sc_gather(flat_tables, flat_idx):
    n = flat_idx.shape[1]
    mesh = plsc.VectorSubcoreMesh(core_axis_name="c", subcore_axis_name="s")

    @functools.partial(
        pl.kernel,
        out_type=jax.ShapeDtypeStruct((n, flat_tables.shape[1]), flat_tables.dtype),
        mesh=mesh,
    )
    def gather_kernel(x_hbm, i_hbm, o_hbm):
        def body(i_vmem, o_vmem):
            pltpu.sync_copy(x_hbm.at[i_vmem.at[0]], o_vmem)

        pltpu.emit_pipeline(
            body,
            grid=(n // _GATHER_WINDOW,),
            in_specs=[pl.BlockSpec((1, _GATHER_WINDOW), index_map=lambda i: (0, i))],
            out_specs=[
                pl.BlockSpec(
                    (_GATHER_WINDOW, flat_tables.shape[1]),
                    index_map=lambda i: (i, 0),
                )
            ],
            core_axis_name=("c", "s"),
            dimension_semantics=(pltpu.PARALLEL,),
        )(i_hbm, o_hbm)

    return gather_kernel(flat_tables, flat_idx)


def _mlp_body(r_ref, w10, b10, w20, b20, w11, b11, w21, b21, w12, b12, w22,
              b22, wd, bd, o_ref):
    r = r_ref[...]
    for w1, b1, w2, b2 in ((w10, b10, w20, b20), (w11, b11, w21, b21),
                           (w12, b12, w22, b22)):
        h = jnp.dot(r.astype(jnp.bfloat16), w1[...],
                    preferred_element_type=jnp.float32) + b1[...]
        h = jnp.maximum(h, 0.0)
        h = jnp.dot(h.astype(jnp.bfloat16), w2[...],
                    preferred_element_type=jnp.float32) + b2[...]
        r = jnp.maximum(r + h, 0.0)
    logits = jnp.dot(r.astype(jnp.bfloat16), wd[...],
                     preferred_element_type=jnp.float32) + bd[...]
    o_ref[...] = jax.nn.sigmoid(logits)


def _mlp(r, weights, block_m, interpret=False):
    n_rows = r.shape[0]
    grid = (n_rows // block_m,)
    full = lambda arr: pl.BlockSpec(arr.shape, lambda i: (0,) * arr.ndim)
    in_specs = [pl.BlockSpec((block_m, L), lambda i: (i, 0))]
    in_specs += [full(w) for w in weights]
    return pl.pallas_call(
        _mlp_body,
        grid=grid,
        in_specs=in_specs,
        out_specs=pl.BlockSpec((block_m, 1), lambda i: (i, 0)),
        out_shape=jax.ShapeDtypeStruct((n_rows, 1), jnp.float32),
        interpret=interpret,
    )(r, *weights)


def kernel(inputs, tables, W1_0, b1_0, W2_0, b2_0, W1_1, b1_1, W2_1, b2_1,
           W1_2, b1_2, W2_2, b2_2, Wd, bd):
    flat_tables = tables.reshape(F * V, D)
    flat_idx = (inputs.astype(jnp.int32)
                + jnp.arange(F, dtype=jnp.int32)[None, :] * V)
    flat_idx = flat_idx.reshape(1, B * F)
    r = _sc_gather(flat_tables, flat_idx).reshape(B, L)

    bf = jnp.bfloat16
    weights = (
        W1_0.astype(bf), b1_0.reshape(1, -1), W2_0.astype(bf), b2_0.reshape(1, -1),
        W1_1.astype(bf), b1_1.reshape(1, -1), W2_1.astype(bf), b2_1.reshape(1, -1),
        W1_2.astype(bf), b1_2.reshape(1, -1), W2_2.astype(bf), b2_2.reshape(1, -1),
        Wd.astype(bf), bd.reshape(1, 1),
    )
    return _mlp(r, weights, block_m=512)


# speedup vs baseline: 10.4734x; 1.0041x over previous
import functools

import jax
import jax.numpy as jnp
from jax.experimental import pallas as pl
from jax.experimental.pallas import tpu as pltpu
from jax.experimental.pallas import tpu_sc as plsc

B = 4096
F = 26
V = 1000
D = 128
L = F * D

_GATHER_ROWS = 128
_GATHER_R = 2


def _sc_gather(flat_tables, idx3):
    s, rr, _ = idx3.shape
    window = rr * _GATHER_ROWS
    n = s * window
    mesh = plsc.VectorSubcoreMesh(core_axis_name="c", subcore_axis_name="s")

    @functools.partial(
        pl.kernel,
        out_type=jax.ShapeDtypeStruct((n, flat_tables.shape[1]), flat_tables.dtype),
        mesh=mesh,
    )
    def gather_kernel(x_hbm, i_hbm, o_hbm):
        def body(i_vmem, o_vmem):
            for j in range(rr):
                pltpu.sync_copy(
                    x_hbm.at[i_vmem.at[0, j]],
                    o_vmem.at[pl.ds(j * _GATHER_ROWS, _GATHER_ROWS)],
                )

        pltpu.emit_pipeline(
            body,
            grid=(s,),
            in_specs=[
                pl.BlockSpec((1, rr, _GATHER_ROWS), index_map=lambda i: (i, 0, 0))
            ],
            out_specs=[
                pl.BlockSpec(
                    (window, flat_tables.shape[1]),
                    index_map=lambda i: (i, 0),
                )
            ],
            core_axis_name=("c", "s"),
            dimension_semantics=(pltpu.PARALLEL,),
        )(i_hbm, o_hbm)

    return gather_kernel(flat_tables, idx3)


def _mlp_body(r_ref, w10, b10, w20, b20, w11, b11, w21, b21, w12, b12, w22,
              b22, wd, bd, o_ref):
    r = r_ref[...].astype(jnp.float32)
    for w1, b1, w2, b2 in ((w10, b10, w20, b20), (w11, b11, w21, b21),
                           (w12, b12, w22, b22)):
        h = jnp.dot(r.astype(jnp.bfloat16), w1[...],
                    preferred_element_type=jnp.float32) + b1[...]
        h = jnp.maximum(h, 0.0)
        h = jnp.dot(h.astype(jnp.bfloat16), w2[...],
                    preferred_element_type=jnp.float32) + b2[...]
        r = jnp.maximum(r + h, 0.0)
    logits = jnp.dot(r.astype(jnp.bfloat16), wd[...],
                     preferred_element_type=jnp.float32) + bd[...]
    o_ref[...] = jax.nn.sigmoid(logits)


def _mlp(r, weights, block_m, interpret=False):
    n_rows = r.shape[0]
    grid = (n_rows // block_m,)
    full = lambda arr: pl.BlockSpec(arr.shape, lambda i: (0,) * arr.ndim)
    in_specs = [pl.BlockSpec((block_m, L), lambda i: (i, 0))]
    in_specs += [full(w) for w in weights]
    return pl.pallas_call(
        _mlp_body,
        grid=grid,
        in_specs=in_specs,
        out_specs=pl.BlockSpec((block_m, 1), lambda i: (i, 0)),
        out_shape=jax.ShapeDtypeStruct((n_rows, 1), jnp.float32),
        interpret=interpret,
    )(r, *weights)


def kernel(inputs, tables, W1_0, b1_0, W2_0, b2_0, W1_1, b1_1, W2_1, b2_1,
           W1_2, b1_2, W2_2, b2_2, Wd, bd):
    flat_tables = tables.reshape(F * V, D)
    flat_idx = (inputs.astype(jnp.int32)
                + jnp.arange(F, dtype=jnp.int32)[None, :] * V)
    window = _GATHER_R * _GATHER_ROWS
    flat_idx = flat_idx.reshape(B * F // window, _GATHER_R, _GATHER_ROWS)
    r = _sc_gather(flat_tables, flat_idx).reshape(B, L)

    bf = jnp.bfloat16
    weights = (
        W1_0.astype(bf), b1_0.reshape(1, -1), W2_0.astype(bf), b2_0.reshape(1, -1),
        W1_1.astype(bf), b1_1.reshape(1, -1), W2_1.astype(bf), b2_1.reshape(1, -1),
        W1_2.astype(bf), b1_2.reshape(1, -1), W2_2.astype(bf), b2_2.reshape(1, -1),
        Wd.astype(bf), bd.reshape(1, 1),
    )
    return _mlp(r, weights, block_m=512)


# speedup vs baseline: 14.6651x; 1.4002x over previous
import functools

import jax
import jax.numpy as jnp
from jax.experimental import pallas as pl
from jax.experimental.pallas import tpu as pltpu
from jax.experimental.pallas import tpu_sc as plsc

B = 4096
F = 26
V = 1000
D = 128
L = F * D

_GATHER_ROWS = 128
_GATHER_R = 2


def _sc_gather(flat_tables, idx3):
    s, rr, _ = idx3.shape
    window = rr * _GATHER_ROWS
    n = s * window
    mesh = plsc.VectorSubcoreMesh(core_axis_name="c", subcore_axis_name="s")

    @functools.partial(
        pl.kernel,
        out_type=jax.ShapeDtypeStruct((n, flat_tables.shape[1]), flat_tables.dtype),
        mesh=mesh,
    )
    def gather_kernel(x_hbm, i_hbm, o_hbm):
        def body(i_vmem, o_vmem):
            for j in range(rr):
                pltpu.sync_copy(
                    x_hbm.at[i_vmem.at[0, j]],
                    o_vmem.at[pl.ds(j * _GATHER_ROWS, _GATHER_ROWS)],
                )

        pltpu.emit_pipeline(
            body,
            grid=(s,),
            in_specs=[
                pl.BlockSpec((1, rr, _GATHER_ROWS), index_map=lambda i: (i, 0, 0))
            ],
            out_specs=[
                pl.BlockSpec(
                    (window, flat_tables.shape[1]),
                    index_map=lambda i: (i, 0),
                )
            ],
            core_axis_name=("c", "s"),
            dimension_semantics=(pltpu.PARALLEL,),
        )(i_hbm, o_hbm)

    return gather_kernel(flat_tables, idx3)


def _mlp_body(e_ref, w10, b10, w20, b20, w11, b11, w21, b21, w12, b12, w22,
              b22, wd, bd, o_ref):
    r = jnp.concatenate([e_ref[f] for f in range(F)], axis=1)
    for w1, b1, w2, b2 in ((w10, b10, w20, b20), (w11, b11, w21, b21),
                           (w12, b12, w22, b22)):
        h = jnp.dot(r.astype(jnp.bfloat16), w1[...],
                    preferred_element_type=jnp.float32) + b1[...]
        h = jnp.maximum(h, 0.0)
        h = jnp.dot(h.astype(jnp.bfloat16), w2[...],
                    preferred_element_type=jnp.float32) + b2[...]
        r = jnp.maximum(r + h, 0.0)
    logits = jnp.dot(r.astype(jnp.bfloat16), wd[...],
                     preferred_element_type=jnp.float32) + bd[...]
    o_ref[...] = jax.nn.sigmoid(logits)


def _mlp(emb, weights, block_m, interpret=False):
    n_rows = emb.shape[1]
    grid = (n_rows // block_m,)
    full = lambda arr: pl.BlockSpec(arr.shape, lambda i: (0,) * arr.ndim)
    in_specs = [pl.BlockSpec((F, block_m, D), lambda i: (0, i, 0))]
    in_specs += [full(w) for w in weights]
    return pl.pallas_call(
        _mlp_body,
        grid=grid,
        in_specs=in_specs,
        out_specs=pl.BlockSpec((block_m, 1), lambda i: (i, 0)),
        out_shape=jax.ShapeDtypeStruct((n_rows, 1), jnp.float32),
        interpret=interpret,
    )(emb, *weights)


def kernel(inputs, tables, W1_0, b1_0, W2_0, b2_0, W1_1, b1_1, W2_1, b2_1,
           W1_2, b1_2, W2_2, b2_2, Wd, bd):
    flat_tables = tables.reshape(F * V, D)
    idx_fm = (inputs.astype(jnp.int32).T
              + jnp.arange(F, dtype=jnp.int32)[:, None] * V)
    window = _GATHER_R * _GATHER_ROWS
    idx3 = idx_fm.reshape(F * B // window, _GATHER_R, _GATHER_ROWS)
    emb = _sc_gather(flat_tables, idx3).reshape(F, B, D)

    bf = jnp.bfloat16
    weights = (
        W1_0.astype(bf), b1_0.reshape(1, -1), W2_0.astype(bf), b2_0.reshape(1, -1),
        W1_1.astype(bf), b1_1.reshape(1, -1), W2_1.astype(bf), b2_1.reshape(1, -1),
        W1_2.astype(bf), b1_2.reshape(1, -1), W2_2.astype(bf), b2_2.reshape(1, -1),
        Wd.astype(bf), bd.reshape(1, 1),
    )
    return _mlp(emb, weights, block_m=512)
